# Initial kernel scaffold; baseline (speedup 1.0000x reference)
import jax
import jax.numpy as jnp
from jax.experimental import pallas as pl


def kernel(x, objective, indices, rev_indices):
    raise NotImplementedError("write your pallas kernel here")


# speedup vs baseline: 1.1463x; 1.1463x over previous
import jax
import jax.numpy as jnp
from jax.experimental import pallas as pl

_ROWS = 512
_COLS = 512


def _shuffle_body(idx_ref, x_ref, o_ref):
    chans = x_ref.shape[1]
    idx = idx_ref[0]
    iota = jax.lax.broadcasted_iota(jnp.int32, (chans, _COLS), 0)
    onehot = (iota == idx[None, :]).astype(jnp.float32)
    o_ref[...] = jnp.dot(x_ref[...], onehot,
                         preferred_element_type=jnp.float32)


@jax.jit
def _shuffle(x, indices):
    batch, chans = x.shape
    idx2d = indices.reshape(1, chans)
    grid = (batch // _ROWS, chans // _COLS)
    return pl.pallas_call(
        _shuffle_body,
        grid=grid,
        in_specs=[
            pl.BlockSpec((1, _COLS), lambda i, j: (0, j)),
            pl.BlockSpec((_ROWS, chans), lambda i, j: (i, 0)),
        ],
        out_specs=pl.BlockSpec((_ROWS, _COLS), lambda i, j: (i, j)),
        out_shape=jax.ShapeDtypeStruct((batch, chans), x.dtype),
    )(idx2d, x)


def kernel(x, objective, indices, rev_indices):
    return (_shuffle(x, indices), objective)
